# baseline (speedup 1.0000x reference)
import functools

import jax
import jax.numpy as jnp
from jax import lax
from jax.experimental import pallas as pl
from jax.experimental.pallas import tpu as pltpu
from jax.experimental.pallas import tpu_sc as plsc

_CHUNK = 128
_DP = 128


@functools.lru_cache(maxsize=None)
def _make_gather_kernel(B, D, NC, NS):
    NW = NC * NS
    b_per_w = B // NW
    n_chunks = b_per_w // _CHUNK
    mesh = plsc.VectorSubcoreMesh(core_axis_name="c", subcore_axis_name="s")

    @functools.partial(
        pl.kernel,
        mesh=mesh,
        out_type=jax.ShapeDtypeStruct((B, _DP), jnp.float32),
        scratch_types=[
            pltpu.VMEM((n_chunks, _CHUNK), jnp.int32),
            pltpu.VMEM((b_per_w, _DP), jnp.float32),
            pltpu.SemaphoreType.DMA,
        ],
    )
    def gather_scale(idx_hbm, tab_hbm, out_hbm, idx_v, rows_v, sem):
        wid = lax.axis_index("s") * NC + lax.axis_index("c")
        base = wid * b_per_w
        for j in range(n_chunks):
            pltpu.sync_copy(idx_hbm.at[pl.ds(base + j * _CHUNK, _CHUNK)],
                            idx_v.at[j])
        copies = [
            pltpu.async_copy(
                tab_hbm.at[idx_v.at[j]],
                rows_v.at[pl.ds(j * _CHUNK, _CHUNK)], sem)
            for j in range(n_chunks)
        ]
        for c in copies:
            c.wait()

        def scale_row(r, carry):
            for k in range(D // 16):
                sl = pl.ds(k * 16, 16)
                rows_v[r, sl] = rows_v[r, sl] * 0.25
            return carry

        lax.fori_loop(0, b_per_w, scale_row, 0)
        pltpu.sync_copy(rows_v, out_hbm.at[pl.ds(base, b_per_w)])

    return gather_scale


@functools.lru_cache(maxsize=None)
def _make_widen_kernel(N, D, block_rows):
    def body(in_ref, out_ref):
        x = in_ref[...]
        out_ref[:, :D] = x
        out_ref[:, D:] = x

    return pl.pallas_call(
        body,
        grid=(N // block_rows,),
        in_specs=[pl.BlockSpec((block_rows, D), lambda i: (i, 0))],
        out_specs=pl.BlockSpec((block_rows, 2 * D), lambda i: (i, 0)),
        out_shape=jax.ShapeDtypeStruct((N, 2 * D), jnp.float32),
    )


def kernel(users, items, user_table, item_table, edge_user, edge_item):
    B = users.shape[0]
    N, D = user_table.shape
    info = plsc.get_sparse_core_info()
    fn = _make_gather_kernel(B, D, info.num_cores, info.num_subcores)
    widen = _make_widen_kernel(N, D, 2000)
    out_u = fn(users, widen(user_table))
    out_i = fn(items, widen(item_table))
    return out_u[:, :D], out_i[:, :D]
